# baseline (speedup 1.0000x reference)
import functools

import jax
import jax.numpy as jnp
from jax import lax
from jax.experimental import pallas as pl
from jax.experimental.pallas import tpu as pltpu
from jax.experimental.pallas import tpu_sc as plsc

_NC = 2
_NS = 16
_LANES = 16


def _make_sc_pool(B, L, V, D):
    NW = _NC * _NS
    assert B % NW == 0
    b_per_w = B // NW
    n_half = 2
    assert L % n_half == 0
    half = L // n_half
    assert half <= 128
    nvec = D // _LANES

    mesh = plsc.VectorSubcoreMesh(core_axis_name="c", subcore_axis_name="s")

    @functools.partial(
        pl.kernel,
        out_type=jax.ShapeDtypeStruct((B, D), jnp.float32),
        mesh=mesh,
        compiler_params=pltpu.CompilerParams(use_tc_tiling_on_sc=False),
        scratch_types=[
            pltpu.VMEM((n_half * b_per_w, half), jnp.int32),
            pltpu.VMEM((half, D), jnp.float32),
            pltpu.VMEM((half, D), jnp.float32),
            pltpu.VMEM((half, D), jnp.float32),
            pltpu.VMEM((half, D), jnp.float32),
            pltpu.VMEM((b_per_w, D), jnp.float32),
            pltpu.SemaphoreType.DMA,
            pltpu.SemaphoreType.DMA,
            pltpu.SemaphoreType.DMA,
            pltpu.SemaphoreType.DMA,
        ],
    )
    def sc_pool(x_hbm, table_hbm, out_hbm, idx_v, buf_a0, buf_a1, buf_b0,
                buf_b1, pooled_v, sem_a0, sem_a1, sem_b0, sem_b1):
        wid = lax.axis_index("s") * _NC + lax.axis_index("c")
        base = wid * b_per_w
        pltpu.sync_copy(x_hbm.at[pl.ds(base * n_half, n_half * b_per_w)],
                        idx_v)

        def start(elem, b0, b1, s0, s1):
            h0 = pltpu.async_copy(table_hbm.at[idx_v.at[n_half * elem]], b0,
                                  s0)
            h1 = pltpu.async_copy(table_hbm.at[idx_v.at[n_half * elem + 1]],
                                  b1, s1)
            return h0, h1

        def wait_for(buf, sem):
            pltpu.make_async_copy(table_hbm.at[idx_v.at[0]], buf, sem).wait()

        def accum(buf, acc):
            def rbody(r, a):
                return tuple(a[j] + buf[r, pl.ds(j * _LANES, _LANES)]
                             for j in range(nvec))

            return lax.fori_loop(0, half, rbody, acc, unroll=4)

        def store_row(row, acc):
            for j in range(nvec):
                pooled_v[row, pl.ds(j * _LANES, _LANES)] = acc[j]

        zeros = tuple(
            jnp.zeros((_LANES,), jnp.float32) for _ in range(nvec))

        start(0, buf_a0, buf_a1, sem_a0, sem_a1)

        def body(j, carry):
            e0 = 2 * j
            hb0, hb1 = start(e0 + 1, buf_b0, buf_b1, sem_b0, sem_b1)
            wait_for(buf_a0, sem_a0)
            acc = accum(buf_a0, zeros)
            wait_for(buf_a1, sem_a1)
            acc = accum(buf_a1, acc)
            store_row(e0, acc)

            @pl.when(j + 1 < b_per_w // 2)
            def _():
                start(e0 + 2, buf_a0, buf_a1, sem_a0, sem_a1)

            hb0.wait()
            acc = accum(buf_b0, zeros)
            hb1.wait()
            acc = accum(buf_b1, acc)
            store_row(e0 + 1, acc)
            return carry

        lax.fori_loop(0, b_per_w // 2, body, 0)
        pltpu.sync_copy(pooled_v, out_hbm.at[pl.ds(base, b_per_w)])

    return sc_pool


def _lin_body(p_ref, w_ref, b_ref, o_ref):
    o_ref[...] = (
        jnp.dot(p_ref[...], w_ref[...], preferred_element_type=jnp.float32)
        + b_ref[...])


def kernel(x, embed_weight, lin_w, lin_b):
    B, L = x.shape
    V, D = embed_weight.shape
    C = lin_w.shape[0]

    x2 = x.reshape(B * 2, L // 2)
    pooled = _make_sc_pool(B, L, V, D)(x2, embed_weight)

    logits = pl.pallas_call(
        _lin_body,
        out_shape=jax.ShapeDtypeStruct((B, C), jnp.float32),
    )(pooled, lin_w.T, lin_b.reshape(1, C))
    return logits


# speedup vs baseline: 1.8108x; 1.8108x over previous
import functools

import jax
import jax.numpy as jnp
from jax import lax
from jax.experimental import pallas as pl
from jax.experimental.pallas import tpu as pltpu
from jax.experimental.pallas import tpu_sc as plsc

_NC = 2
_NS = 16
_LANES = 16


def _make_sc_pool(B, L, V, D):
    NW = _NC * _NS
    assert B % NW == 0
    b_per_w = B // NW
    n_half = 2
    assert L % n_half == 0
    half = L // n_half
    assert half <= 128
    nvec = D // _LANES

    mesh = plsc.VectorSubcoreMesh(core_axis_name="c", subcore_axis_name="s")

    @functools.partial(
        pl.kernel,
        out_type=jax.ShapeDtypeStruct((B, D), jnp.float32),
        mesh=mesh,
        compiler_params=pltpu.CompilerParams(use_tc_tiling_on_sc=False),
        scratch_types=[
            pltpu.VMEM((n_half * b_per_w, half), jnp.int32),
            pltpu.VMEM((half, D), jnp.float32),
            pltpu.VMEM((half, D), jnp.float32),
            pltpu.VMEM((half, D), jnp.float32),
            pltpu.VMEM((half, D), jnp.float32),
            pltpu.VMEM((b_per_w, D), jnp.float32),
            pltpu.SemaphoreType.DMA,
            pltpu.SemaphoreType.DMA,
            pltpu.SemaphoreType.DMA,
            pltpu.SemaphoreType.DMA,
        ],
    )
    def sc_pool(x_hbm, table_hbm, out_hbm, idx_v, buf_a0, buf_a1, buf_b0,
                buf_b1, pooled_v, sem_a0, sem_a1, sem_b0, sem_b1):
        wid = lax.axis_index("s") * _NC + lax.axis_index("c")
        base = wid * b_per_w
        pltpu.sync_copy(x_hbm.at[pl.ds(base * n_half, n_half * b_per_w)],
                        idx_v)

        def start(elem, b0, b1, s0, s1):
            h0 = pltpu.async_copy(table_hbm.at[idx_v.at[n_half * elem]], b0,
                                  s0)
            h1 = pltpu.async_copy(table_hbm.at[idx_v.at[n_half * elem + 1]],
                                  b1, s1)
            return h0, h1

        def wait_for(buf, sem):
            pltpu.make_async_copy(table_hbm.at[idx_v.at[0]], buf, sem).wait()

        def accum(buf, acc):
            def rbody(r, a):
                return tuple(a[j] + buf[r, pl.ds(j * _LANES, _LANES)]
                             for j in range(nvec))

            return lax.fori_loop(0, half, rbody, acc, unroll=4)

        def store_row(row, acc):
            for j in range(nvec):
                pooled_v[row, pl.ds(j * _LANES, _LANES)] = acc[j]

        zeros = tuple(
            jnp.zeros((_LANES,), jnp.float32) for _ in range(nvec))

        start(0, buf_a0, buf_a1, sem_a0, sem_a1)

        def body(j, carry):
            e0 = 2 * j
            hb0, hb1 = start(e0 + 1, buf_b0, buf_b1, sem_b0, sem_b1)
            wait_for(buf_a0, sem_a0)
            acc = accum(buf_a0, zeros)
            wait_for(buf_a1, sem_a1)
            acc = accum(buf_a1, acc)
            store_row(e0, acc)

            @pl.when(j + 1 < b_per_w // 2)
            def _():
                start(e0 + 2, buf_a0, buf_a1, sem_a0, sem_a1)

            hb0.wait()
            acc = accum(buf_b0, zeros)
            hb1.wait()
            acc = accum(buf_b1, acc)
            store_row(e0 + 1, acc)
            return carry

        lax.fori_loop(0, b_per_w // 2, body, 0)
        pltpu.sync_copy(pooled_v, out_hbm.at[pl.ds(base, b_per_w)])

    return sc_pool


_G = 4096


def _transpose_body(a_ref, b_ref, o_ref):
    a = jnp.transpose(a_ref[...], (1, 0))
    b = jnp.transpose(b_ref[...], (1, 0))
    t = jnp.concatenate([a, b], axis=1)
    o_ref[...] = t.reshape(o_ref.shape)


def _repack_table(embed_weight):
    V, D = embed_weight.shape
    tT = embed_weight.T
    grid = (V + 2 * _G - 1) // (2 * _G)
    vpad = grid * 2 * _G
    last_blk = (V - 1) // _G
    return pl.pallas_call(
        _transpose_body,
        grid=(grid,),
        in_specs=[
            pl.BlockSpec((D, _G), lambda g: (0, 2 * g)),
            pl.BlockSpec(
                (D, _G),
                lambda g: (0, jnp.minimum(2 * g + 1, last_blk))),
        ],
        out_specs=pl.BlockSpec((2 * _G * D,), lambda g: (g,)),
        out_shape=jax.ShapeDtypeStruct((vpad * D,), jnp.float32),
    )(tT, tT)


def _remap_idx(x):
    blk = (x >> 13) << 13
    return blk + ((x & (_G - 1)) << 1) + ((x >> 12) & 1)


def _lin_body(p_ref, w_ref, b_ref, o_ref):
    o_ref[...] = (
        jnp.dot(p_ref[...], w_ref[...], preferred_element_type=jnp.float32)
        + b_ref[...])


def kernel(x, embed_weight, lin_w, lin_b):
    B, L = x.shape
    V, D = embed_weight.shape
    C = lin_w.shape[0]

    flat = _repack_table(embed_weight)
    vpad = flat.shape[0] // D
    table = flat.reshape(vpad, D)

    x2 = _remap_idx(x).reshape(B * 2, L // 2)
    pooled = _make_sc_pool(B, L, vpad, D)(x2, table)

    logits = pl.pallas_call(
        _lin_body,
        out_shape=jax.ShapeDtypeStruct((B, C), jnp.float32),
    )(pooled, lin_w.T, lin_b.reshape(1, C))
    return logits


# speedup vs baseline: 1.9646x; 1.0849x over previous
import functools

import jax
import jax.numpy as jnp
from jax import lax
from jax.experimental import pallas as pl
from jax.experimental.pallas import tpu as pltpu
from jax.experimental.pallas import tpu_sc as plsc

_NC = 2
_NS = 16
_LANES = 16


def _make_sc_pool(B, L, V, D):
    NW = _NC * _NS
    assert B % NW == 0
    b_per_w = B // NW
    n_half = 2
    assert L % n_half == 0
    half = L // n_half
    assert half <= 128
    nvec = D // _LANES

    mesh = plsc.VectorSubcoreMesh(core_axis_name="c", subcore_axis_name="s")

    _NBUF = 4

    @functools.partial(
        pl.kernel,
        out_type=jax.ShapeDtypeStruct((B, D), jnp.float32),
        mesh=mesh,
        compiler_params=pltpu.CompilerParams(use_tc_tiling_on_sc=False),
        scratch_types=(
            [pltpu.VMEM((n_half * b_per_w, half), jnp.int32)]
            + [pltpu.VMEM((half, D), jnp.float32)] * (2 * _NBUF)
            + [pltpu.VMEM((b_per_w, D), jnp.float32)]
            + [pltpu.SemaphoreType.DMA] * (2 * _NBUF)
        ),
    )
    def sc_pool(x_hbm, table_hbm, out_hbm, idx_v, *rest):
        bufs = rest[:2 * _NBUF]
        pooled_v = rest[2 * _NBUF]
        sems = rest[2 * _NBUF + 1:]
        wid = lax.axis_index("s") * _NC + lax.axis_index("c")
        base = wid * b_per_w
        pltpu.sync_copy(x_hbm.at[pl.ds(base * n_half, n_half * b_per_w)],
                        idx_v)

        def start(elem, k):
            pltpu.async_copy(table_hbm.at[idx_v.at[n_half * elem]],
                             bufs[2 * k], sems[2 * k])
            pltpu.async_copy(table_hbm.at[idx_v.at[n_half * elem + 1]],
                             bufs[2 * k + 1], sems[2 * k + 1])

        def wait_for(i):
            pltpu.make_async_copy(table_hbm.at[idx_v.at[0]], bufs[i],
                                  sems[i]).wait()

        def accum(buf, acc):
            def rbody(r, a):
                return tuple(a[j] + buf[r, pl.ds(j * _LANES, _LANES)]
                             for j in range(nvec))

            return lax.fori_loop(0, half, rbody, acc, unroll=4)

        def store_row(row, acc):
            for j in range(nvec):
                pooled_v[row, pl.ds(j * _LANES, _LANES)] = acc[j]

        zeros = tuple(
            jnp.zeros((_LANES,), jnp.float32) for _ in range(nvec))

        for k in range(_NBUF):
            start(k, k)

        def body(j, carry):
            e0 = _NBUF * j
            for k in range(_NBUF):
                e = e0 + k
                wait_for(2 * k)
                acc = accum(bufs[2 * k], zeros)
                wait_for(2 * k + 1)
                acc = accum(bufs[2 * k + 1], acc)
                store_row(e, acc)

                @pl.when(e + _NBUF < b_per_w)
                def _(e=e, k=k):
                    start(e + _NBUF, k)

            return carry

        lax.fori_loop(0, b_per_w // _NBUF, body, 0)
        pltpu.sync_copy(pooled_v, out_hbm.at[pl.ds(base, b_per_w)])

    return sc_pool


_G = 4096


def _transpose_body(a_ref, b_ref, o_ref):
    d = a_ref.shape[0]
    eye = jnp.eye(d, dtype=jnp.float32)
    dims = (((0,), (0,)), ((), ()))
    a = jax.lax.dot_general(a_ref[...], eye, dims,
                            preferred_element_type=jnp.float32)
    b = jax.lax.dot_general(b_ref[...], eye, dims,
                            preferred_element_type=jnp.float32)
    o_ref[...] = jnp.concatenate([a, b], axis=1)


def _repack_table(embed_weight):
    V, D = embed_weight.shape
    tT = embed_weight.T
    grid = (V + 2 * _G - 1) // (2 * _G)
    vpad = grid * 2 * _G
    last_blk = (V - 1) // _G
    return pl.pallas_call(
        _transpose_body,
        grid=(grid,),
        in_specs=[
            pl.BlockSpec((D, _G), lambda g: (0, 2 * g)),
            pl.BlockSpec(
                (D, _G),
                lambda g: (0, jnp.minimum(2 * g + 1, last_blk))),
        ],
        out_specs=pl.BlockSpec((_G, 2 * D), lambda g: (g, 0)),
        out_shape=jax.ShapeDtypeStruct((vpad // 2, 2 * D), jnp.float32),
    )(tT, tT)


def _remap_idx(x):
    blk = (x >> 13) << 13
    return blk + ((x & (_G - 1)) << 1) + ((x >> 12) & 1)


def _lin_body(p_ref, w_ref, b_ref, o_ref):
    o_ref[...] = (
        jnp.dot(p_ref[...], w_ref[...], preferred_element_type=jnp.float32)
        + b_ref[...])


def kernel(x, embed_weight, lin_w, lin_b):
    B, L = x.shape
    V, D = embed_weight.shape
    C = lin_w.shape[0]

    packed = _repack_table(embed_weight)
    vpad = packed.shape[0] * 2
    table = packed.reshape(vpad, D)

    x2 = _remap_idx(x).reshape(B * 2, L // 2)
    pooled = _make_sc_pool(B, L, vpad, D)(x2, table)

    logits = pl.pallas_call(
        _lin_body,
        out_shape=jax.ShapeDtypeStruct((B, C), jnp.float32),
    )(pooled, lin_w.T, lin_b.reshape(1, C))
    return logits


# speedup vs baseline: 1.9679x; 1.0017x over previous
import functools

import jax
import jax.numpy as jnp
from jax import lax
from jax.experimental import pallas as pl
from jax.experimental.pallas import tpu as pltpu
from jax.experimental.pallas import tpu_sc as plsc

_NC = 2
_NS = 16
_LANES = 16


def _make_sc_pool(B, L, V, D):
    NW = _NC * _NS
    assert B % NW == 0
    b_per_w = B // NW
    n_half = 2
    assert L % n_half == 0
    half = L // n_half
    assert half <= 128
    nvec = D // _LANES

    mesh = plsc.VectorSubcoreMesh(core_axis_name="c", subcore_axis_name="s")

    _NBUF = 4

    @functools.partial(
        pl.kernel,
        out_type=jax.ShapeDtypeStruct((B, D), jnp.float32),
        mesh=mesh,
        compiler_params=pltpu.CompilerParams(use_tc_tiling_on_sc=False),
        scratch_types=(
            [pltpu.VMEM((n_half * b_per_w, half), jnp.int32)]
            + [pltpu.VMEM((half, D), jnp.float32)] * (2 * _NBUF)
            + [pltpu.VMEM((b_per_w, D), jnp.float32)]
            + [pltpu.SemaphoreType.DMA] * (2 * _NBUF)
        ),
    )
    def sc_pool(x_hbm, table_hbm, out_hbm, idx_v, *rest):
        bufs = rest[:2 * _NBUF]
        pooled_v = rest[2 * _NBUF]
        sems = rest[2 * _NBUF + 1:]
        wid = lax.axis_index("s") * _NC + lax.axis_index("c")
        base = wid * b_per_w
        pltpu.sync_copy(x_hbm.at[pl.ds(base * n_half, n_half * b_per_w)],
                        idx_v)

        def start(elem, k):
            pltpu.async_copy(table_hbm.at[idx_v.at[n_half * elem]],
                             bufs[2 * k], sems[2 * k])
            pltpu.async_copy(table_hbm.at[idx_v.at[n_half * elem + 1]],
                             bufs[2 * k + 1], sems[2 * k + 1])

        def wait_for(i):
            pltpu.make_async_copy(table_hbm.at[idx_v.at[0]], bufs[i],
                                  sems[i]).wait()

        def accum(buf, acc):
            def rbody(r, a):
                return tuple(a[j] + buf[r, pl.ds(j * _LANES, _LANES)]
                             for j in range(nvec))

            return lax.fori_loop(0, half, rbody, acc, unroll=4)

        def store_row(row, acc):
            for j in range(nvec):
                pooled_v[row, pl.ds(j * _LANES, _LANES)] = acc[j]

        zeros = tuple(
            jnp.zeros((_LANES,), jnp.float32) for _ in range(nvec))

        for k in range(_NBUF):
            start(k, k)

        def body(j, carry):
            e0 = _NBUF * j
            for k in range(_NBUF):
                e = e0 + k
                wait_for(2 * k)
                acc = accum(bufs[2 * k], zeros)
                wait_for(2 * k + 1)
                acc = accum(bufs[2 * k + 1], acc)
                store_row(e, acc)

                @pl.when(e + _NBUF < b_per_w)
                def _(e=e, k=k):
                    start(e + _NBUF, k)

            return carry

        lax.fori_loop(0, b_per_w // _NBUF, body, 0)
        pltpu.sync_copy(pooled_v, out_hbm.at[pl.ds(base, b_per_w)])

    return sc_pool


_G = 4096


def _transpose_body(a_ref, b_ref, o_ref):
    a = jnp.transpose(a_ref[...], (1, 0))
    b = jnp.transpose(b_ref[...], (1, 0))
    o_ref[...] = jnp.concatenate([a, b], axis=1)


def _repack_table(embed_weight):
    V, D = embed_weight.shape
    tT = embed_weight.T
    grid = (V + 2 * _G - 1) // (2 * _G)
    vpad = grid * 2 * _G
    last_blk = (V - 1) // _G
    return pl.pallas_call(
        _transpose_body,
        grid=(grid,),
        in_specs=[
            pl.BlockSpec((D, _G), lambda g: (0, 2 * g)),
            pl.BlockSpec(
                (D, _G),
                lambda g: (0, jnp.minimum(2 * g + 1, last_blk))),
        ],
        out_specs=pl.BlockSpec((_G, 2 * D), lambda g: (g, 0)),
        out_shape=jax.ShapeDtypeStruct((vpad // 2, 2 * D), jnp.float32),
    )(tT, tT)


def _remap_idx(x):
    blk = (x >> 13) << 13
    return blk + ((x & (_G - 1)) << 1) + ((x >> 12) & 1)


def _lin_body(p_ref, w_ref, b_ref, o_ref):
    o_ref[...] = (
        jnp.dot(p_ref[...], w_ref[...], preferred_element_type=jnp.float32)
        + b_ref[...])


def kernel(x, embed_weight, lin_w, lin_b):
    B, L = x.shape
    V, D = embed_weight.shape
    C = lin_w.shape[0]

    packed = _repack_table(embed_weight)
    vpad = packed.shape[0] * 2
    table = packed.reshape(vpad, D)

    x2 = _remap_idx(x).reshape(B * 2, L // 2)
    pooled = _make_sc_pool(B, L, vpad, D)(x2, table)

    logits = pl.pallas_call(
        _lin_body,
        out_shape=jax.ShapeDtypeStruct((B, C), jnp.float32),
    )(pooled, lin_w.T, lin_b.reshape(1, C))
    return logits


# speedup vs baseline: 2.1727x; 1.1040x over previous
import functools

import jax
import jax.numpy as jnp
from jax import lax
from jax.experimental import pallas as pl
from jax.experimental.pallas import tpu as pltpu
from jax.experimental.pallas import tpu_sc as plsc

_NC = 2
_NS = 16
_LANES = 16


def _make_sc_pool(B, L, V, D):
    NW = _NC * _NS
    assert B % NW == 0
    b_per_w = B // NW
    n_half = 2
    assert L % n_half == 0
    half = L // n_half
    assert half <= 128
    nvec = D // _LANES

    mesh = plsc.VectorSubcoreMesh(core_axis_name="c", subcore_axis_name="s")

    _NBUF = 4

    @functools.partial(
        pl.kernel,
        out_type=jax.ShapeDtypeStruct((B, D), jnp.float32),
        mesh=mesh,
        compiler_params=pltpu.CompilerParams(use_tc_tiling_on_sc=False),
        scratch_types=(
            [pltpu.VMEM((n_half * b_per_w, half), jnp.int32)]
            + [pltpu.VMEM((half, D), jnp.float32)] * (2 * _NBUF)
            + [pltpu.VMEM((b_per_w, D), jnp.float32)]
            + [pltpu.SemaphoreType.DMA] * (2 * _NBUF)
        ),
    )
    def sc_pool(x_hbm, table_hbm, out_hbm, idx_v, *rest):
        bufs = rest[:2 * _NBUF]
        pooled_v = rest[2 * _NBUF]
        sems = rest[2 * _NBUF + 1:]
        wid = lax.axis_index("s") * _NC + lax.axis_index("c")
        base = wid * b_per_w
        pltpu.sync_copy(x_hbm.at[pl.ds(base * n_half, n_half * b_per_w)],
                        idx_v)

        def start(elem, k):
            pltpu.async_copy(table_hbm.at[idx_v.at[n_half * elem]],
                             bufs[2 * k], sems[2 * k])
            pltpu.async_copy(table_hbm.at[idx_v.at[n_half * elem + 1]],
                             bufs[2 * k + 1], sems[2 * k + 1])

        def wait_for(i):
            pltpu.make_async_copy(table_hbm.at[idx_v.at[0]], bufs[i],
                                  sems[i]).wait()

        def accum(buf, acc):
            def rbody(r, a):
                return tuple(a[j] + buf[r, pl.ds(j * _LANES, _LANES)]
                             for j in range(nvec))

            return lax.fori_loop(0, half, rbody, acc, unroll=4)

        def store_row(row, acc):
            for j in range(nvec):
                pooled_v[row, pl.ds(j * _LANES, _LANES)] = acc[j]

        zeros = tuple(
            jnp.zeros((_LANES,), jnp.float32) for _ in range(nvec))

        for k in range(_NBUF):
            start(k, k)

        def body(j, carry):
            e0 = _NBUF * j
            for k in range(_NBUF):
                e = e0 + k
                wait_for(2 * k)
                acc = accum(bufs[2 * k], zeros)
                wait_for(2 * k + 1)
                acc = accum(bufs[2 * k + 1], acc)
                store_row(e, acc)

                @pl.when(e + _NBUF < b_per_w)
                def _(e=e, k=k):
                    start(e + _NBUF, k)

            return carry

        lax.fori_loop(0, b_per_w // _NBUF, body, 0)
        pltpu.sync_copy(pooled_v, out_hbm.at[pl.ds(base, b_per_w)])

    return sc_pool


_G = 8192
_GSH = 13


def _transpose_body(a_ref, b_ref, o_ref):
    a = jnp.transpose(a_ref[...], (1, 0))
    b = jnp.transpose(b_ref[...], (1, 0))
    o_ref[...] = jnp.concatenate([a, b], axis=1)


def _repack_table(embed_weight):
    V, D = embed_weight.shape
    tT = embed_weight.T
    grid = (V + 2 * _G - 1) // (2 * _G)
    vpad = grid * 2 * _G
    last_blk = (V - 1) // _G
    return pl.pallas_call(
        _transpose_body,
        grid=(grid,),
        in_specs=[
            pl.BlockSpec((D, _G), lambda g: (0, 2 * g)),
            pl.BlockSpec(
                (D, _G),
                lambda g: (0, jnp.minimum(2 * g + 1, last_blk))),
        ],
        out_specs=pl.BlockSpec((_G, 2 * D), lambda g: (g, 0)),
        out_shape=jax.ShapeDtypeStruct((vpad // 2, 2 * D), jnp.float32),
    )(tT, tT)


def _remap_idx(x):
    blk = (x >> (_GSH + 1)) << (_GSH + 1)
    return blk + ((x & (_G - 1)) << 1) + ((x >> _GSH) & 1)


def _lin_body(p_ref, w_ref, b_ref, o_ref):
    o_ref[...] = (
        jnp.dot(p_ref[...], w_ref[...], preferred_element_type=jnp.float32)
        + b_ref[...])


def kernel(x, embed_weight, lin_w, lin_b):
    B, L = x.shape
    V, D = embed_weight.shape
    C = lin_w.shape[0]

    packed = _repack_table(embed_weight)
    vpad = packed.shape[0] * 2
    table = packed.reshape(vpad, D)

    x2 = _remap_idx(x).reshape(B * 2, L // 2)
    pooled = _make_sc_pool(B, L, vpad, D)(x2, table)

    logits = pl.pallas_call(
        _lin_body,
        out_shape=jax.ShapeDtypeStruct((B, C), jnp.float32),
    )(pooled, lin_w.T, lin_b.reshape(1, C))
    return logits


# speedup vs baseline: 2.2584x; 1.0395x over previous
import functools

import jax
import jax.numpy as jnp
from jax import lax
from jax.experimental import pallas as pl
from jax.experimental.pallas import tpu as pltpu
from jax.experimental.pallas import tpu_sc as plsc

_NC = 2
_NS = 16
_LANES = 16


def _make_sc_pool(B, L, V, D):
    NW = _NC * _NS
    assert B % NW == 0
    b_per_w = B // NW
    n_half = 2
    assert L % n_half == 0
    half = L // n_half
    assert half <= 128
    nvec = D // _LANES

    mesh = plsc.VectorSubcoreMesh(core_axis_name="c", subcore_axis_name="s")

    _NBUF = 4

    @functools.partial(
        pl.kernel,
        out_type=jax.ShapeDtypeStruct((B, D), jnp.float32),
        mesh=mesh,
        compiler_params=pltpu.CompilerParams(use_tc_tiling_on_sc=False),
        scratch_types=(
            [pltpu.VMEM((n_half * b_per_w, half), jnp.int32)]
            + [pltpu.VMEM((half, D), jnp.float32)] * (2 * _NBUF)
            + [pltpu.VMEM((b_per_w, D), jnp.float32)]
            + [pltpu.SemaphoreType.DMA] * (2 * _NBUF)
        ),
    )
    def sc_pool(x_hbm, table_hbm, out_hbm, idx_v, *rest):
        bufs = rest[:2 * _NBUF]
        pooled_v = rest[2 * _NBUF]
        sems = rest[2 * _NBUF + 1:]
        wid = lax.axis_index("s") * _NC + lax.axis_index("c")
        base = wid * b_per_w
        pltpu.sync_copy(x_hbm.at[pl.ds(base * n_half, n_half * b_per_w)],
                        idx_v)

        def start(elem, k):
            pltpu.async_copy(table_hbm.at[idx_v.at[n_half * elem]],
                             bufs[2 * k], sems[2 * k])
            pltpu.async_copy(table_hbm.at[idx_v.at[n_half * elem + 1]],
                             bufs[2 * k + 1], sems[2 * k + 1])

        def wait_for(i):
            pltpu.make_async_copy(table_hbm.at[idx_v.at[0]], bufs[i],
                                  sems[i]).wait()

        def accum(buf, acc):
            def rbody(r, a):
                return tuple(a[j] + buf[r, pl.ds(j * _LANES, _LANES)]
                             for j in range(nvec))

            return lax.fori_loop(0, half, rbody, acc, unroll=4)

        def store_row(row, acc):
            for j in range(nvec):
                pooled_v[row, pl.ds(j * _LANES, _LANES)] = acc[j]

        zeros = tuple(
            jnp.zeros((_LANES,), jnp.float32) for _ in range(nvec))

        for k in range(_NBUF):
            start(k, k)

        def body(j, carry):
            e0 = _NBUF * j
            for k in range(_NBUF):
                e = e0 + k
                wait_for(2 * k)
                acc = accum(bufs[2 * k], zeros)
                wait_for(2 * k + 1)
                acc = accum(bufs[2 * k + 1], acc)
                store_row(e, acc)

                @pl.when(e + _NBUF < b_per_w)
                def _(e=e, k=k):
                    start(e + _NBUF, k)

            return carry

        lax.fori_loop(0, b_per_w // _NBUF, body, 0)
        pltpu.sync_copy(pooled_v, out_hbm.at[pl.ds(base, b_per_w)])

    return sc_pool


_G = 16384
_GSH = 14


def _transpose_body(a_ref, b_ref, o_ref):
    a = jnp.transpose(a_ref[...], (1, 0))
    b = jnp.transpose(b_ref[...], (1, 0))
    o_ref[...] = jnp.concatenate([a, b], axis=1)


def _repack_table(embed_weight):
    V, D = embed_weight.shape
    tT = embed_weight.T
    grid = (V + 2 * _G - 1) // (2 * _G)
    vpad = grid * 2 * _G
    last_blk = (V - 1) // _G
    return pl.pallas_call(
        _transpose_body,
        grid=(grid,),
        in_specs=[
            pl.BlockSpec((D, _G), lambda g: (0, 2 * g)),
            pl.BlockSpec(
                (D, _G),
                lambda g: (0, jnp.minimum(2 * g + 1, last_blk))),
        ],
        out_specs=pl.BlockSpec((_G, 2 * D), lambda g: (g, 0)),
        out_shape=jax.ShapeDtypeStruct((vpad // 2, 2 * D), jnp.float32),
    )(tT, tT)


def _remap_idx(x):
    blk = (x >> (_GSH + 1)) << (_GSH + 1)
    return blk + ((x & (_G - 1)) << 1) + ((x >> _GSH) & 1)


def _lin_body(p_ref, w_ref, b_ref, o_ref):
    o_ref[...] = (
        jnp.dot(p_ref[...], w_ref[...], preferred_element_type=jnp.float32)
        + b_ref[...])


def kernel(x, embed_weight, lin_w, lin_b):
    B, L = x.shape
    V, D = embed_weight.shape
    C = lin_w.shape[0]

    packed = _repack_table(embed_weight)
    vpad = packed.shape[0] * 2
    table = packed.reshape(vpad, D)

    x2 = _remap_idx(x).reshape(B * 2, L // 2)
    pooled = _make_sc_pool(B, L, vpad, D)(x2, table)

    logits = pl.pallas_call(
        _lin_body,
        out_shape=jax.ShapeDtypeStruct((B, C), jnp.float32),
    )(pooled, lin_w.T, lin_b.reshape(1, C))
    return logits
